# baseline (speedup 1.0000x reference)
import functools

import jax
import jax.numpy as jnp
from jax import lax
from jax.experimental import pallas as pl
from jax.experimental.pallas import tpu as pltpu
from jax.experimental.pallas import tpu_sc as plsc

_B, _A, _G = 8, 16384, 24
_R = 128
_C = 128
_IOU_TH, _NEG_POS_RATIO, _CONF_W, _LOC_W, _BETA = 0.5, 3, 2.0, 1.0, 0.05
_NV = _A // 16
_UNROLL = 8


def _smooth_l1(d):
    ad = jnp.abs(d)
    return jnp.where(ad < _BETA, 0.5 * ad * ad / _BETA, ad - 0.5 * _BETA)


def _dense_body(px1, px2, py1, py2, conf, ax1, ax2, ay1, ay2, gt_ref,
                nbits_ref, scal_ref):
    iota = (lax.broadcasted_iota(jnp.int32, (_R, _C), 0) * _C
            + lax.broadcasted_iota(jnp.int32, (_R, _C), 1))
    a1 = ax1[...]
    a2 = ax2[...]
    b1 = ay1[...]
    b2 = ay2[...]
    area_a = (a2 - a1) * (b2 - b1)

    for i in range(_B):
        x1 = px1[i]
        x2 = px2[i]
        y1 = py1[i]
        y2 = py2[i]
        p = conf[i]
        bsum = jnp.sum(x1) + jnp.sum(x2) + jnp.sum(y1) + jnp.sum(y2)
        skip = (bsum == 0.0) | (jnp.max(p) < 0.01)

        def g_body(g, carry):
            best, mcx, mcy, mw, mh, force = carry
            gx1 = gt_ref[i, g, 0]
            gy1 = gt_ref[i, g, 1]
            gx2 = gt_ref[i, g, 2]
            gy2 = gt_ref[i, g, 3]
            whx = jnp.clip(jnp.minimum(a2, gx2) - jnp.maximum(a1, gx1), 0.0, None)
            why = jnp.clip(jnp.minimum(b2, gy2) - jnp.maximum(b1, gy1), 0.0, None)
            inter = whx * why
            area_b = (gx2 - gx1) * (gy2 - gy1)
            union = area_a + area_b - inter
            iou = inter / jnp.maximum(union, 1e-9)
            upd = iou > best
            best = jnp.where(upd, iou, best)
            mcx = jnp.where(upd, (gx1 + gx2) * 0.5, mcx)
            mcy = jnp.where(upd, (gy1 + gy2) * 0.5, mcy)
            mw = jnp.where(upd, gx2 - gx1, mw)
            mh = jnp.where(upd, gy2 - gy1, mh)
            m = jnp.max(iou)
            aidx = jnp.min(jnp.where(iou == m, iota, jnp.int32(_A)))
            force = jnp.maximum(force, jnp.where(iota == aidx, 1.0, 0.0))
            return best, mcx, mcy, mw, mh, force

        zero = jnp.zeros((_R, _C), jnp.float32)
        best, mcx, mcy, mw, mh, force = lax.fori_loop(
            0, _G, g_body,
            (jnp.full((_R, _C), -1.0, jnp.float32), zero, zero, zero, zero,
             zero))

        pos = (best > _IOU_TH) | (force > 0.0)
        posf = pos.astype(jnp.float32)
        np_f = jnp.sum(posf)
        np_i = np_f.astype(jnp.int32)

        ll = (_smooth_l1((x1 + x2) * 0.5 - mcx)
              + _smooth_l1((y1 + y2) * 0.5 - mcy)
              + _smooth_l1((x2 - x1) - mw)
              + _smooth_l1((y2 - y1) - mh))
        loc_i = jnp.sum(ll * posf)

        logp = jnp.maximum(jnp.log(p), -100.0)
        log1mp = jnp.maximum(jnp.log(1.0 - p), -100.0)
        pos_conf = jnp.sum(posf * (-logp))
        neg = jnp.abs(jnp.where(pos, 0.0, -log1mp))
        k = jnp.minimum(np_i * _NEG_POS_RATIO, _A - np_i)

        nbits_ref[i] = lax.bitcast_convert_type(neg, jnp.int32)
        vals = (np_f, k.astype(jnp.float32), loc_i, pos_conf,
                jnp.where(skip, jnp.float32(1.0), jnp.float32(0.0)))
        for f in range(5):
            for j in range(16):
                scal_ref[i * 5 + f, j] = vals[f]


_sc_mesh = plsc.VectorSubcoreMesh(core_axis_name="c", subcore_axis_name="s")

_GD = lax.GatherDimensionNumbers(
    offset_dims=(), collapsed_slice_dims=(0,), start_index_map=(0,))


def _shuf(x, perm):
    return lax.gather(x, perm[:, None], _GD, (1,),
                      mode=lax.GatherScatterMode.PROMISE_IN_BOUNDS)


def _splat_sum(x, iota16):
    for sh in (8, 4, 2, 1):
        x = x + _shuf(x, iota16 ^ sh)
    return x


@functools.partial(
    pl.kernel,
    mesh=_sc_mesh,
    compiler_params=pltpu.CompilerParams(needs_layout_passes=False),
    out_type=jax.ShapeDtypeStruct((16,), jnp.float32),
    scratch_types=[
        pltpu.VMEM((_A,), jnp.int32),
        pltpu.VMEM((80,), jnp.float32),
        pltpu.VMEM((16,), jnp.float32),
        pltpu.VMEM((_B * 16,), jnp.float32),
        pltpu.VMEM((_B * 80,), jnp.float32),
        pltpu.VMEM((16,), jnp.float32),
        pltpu.VMEM_SHARED((_B * 16,), jnp.float32),
    ],
)
def _sc_mine(nbits_hbm, scal_hbm, out_hbm, row_v, scal5_v, stage_v, gath_v,
             allscal_v, out_stage, shared):
    c = lax.axis_index("c")
    s = lax.axis_index("s")
    is_worker = (c == 0) & (s < _B)
    iota16 = lax.iota(jnp.int32, 16)
    zf = jnp.zeros((16,), jnp.float32)
    zi = jnp.zeros((16,), jnp.int32)

    @pl.when(is_worker)
    def _worker():
        pltpu.sync_copy(nbits_hbm.at[pl.ds(s * _A, _A)], row_v)
        pltpu.sync_copy(scal_hbm.at[pl.ds(s * 80, 80)], scal5_v)
        kf = scal5_v[pl.ds(16, 16)]
        ki = kf.astype(jnp.int32)

        def bit_body(j, t):
            cand = t | (zi + (jnp.int32(1) << (30 - j)))

            def cnt_body(jj, acc):
                for u in range(_UNROLL):
                    v = row_v[pl.ds(jj * (16 * _UNROLL) + u * 16, 16)]
                    acc = acc + jnp.where(v >= cand, 1, 0).astype(jnp.int32)
                return acc

            cntl = lax.fori_loop(0, _NV // _UNROLL, cnt_body, zi)
            cnt = _splat_sum(cntl, iota16)
            return jnp.where(cnt >= ki, cand, t)

        t_bits = lax.fori_loop(0, 31, bit_body, zi)
        t_val = lax.bitcast_convert_type(t_bits, jnp.float32)

        def sum_body(jj, carry):
            sg, cg = carry
            for u in range(_UNROLL):
                v = row_v[pl.ds(jj * (16 * _UNROLL) + u * 16, 16)]
                m = v > t_bits
                f = lax.bitcast_convert_type(v, jnp.float32)
                sg = sg + jnp.where(m, f, 0.0)
                cg = cg + jnp.where(m, 1.0, 0.0)
            return sg, cg

        sgl, cgl = lax.fori_loop(0, _NV // _UNROLL, sum_body, (zf, zf))
        sum_gt = _splat_sum(sgl, iota16)
        cnt_gt = _splat_sum(cgl, iota16)
        neg_conf = jnp.where(ki > 0, sum_gt + (kf - cnt_gt) * t_val, zf)
        stage_v[...] = neg_conf
        pltpu.sync_copy(stage_v, shared.at[pl.ds(s * 16, 16)])

    plsc.subcore_barrier()

    @pl.when((c == 0) & (s == 0))
    def _leader():
        pltpu.sync_copy(shared, gath_v)
        pltpu.sync_copy(scal_hbm, allscal_v)
        total_loc = zf
        total_conf = zf
        num_pos = zf
        for i in range(_B):
            np_f = allscal_v[pl.ds(i * 80, 16)]
            kf = allscal_v[pl.ds(i * 80 + 16, 16)]
            loc_i = allscal_v[pl.ds(i * 80 + 32, 16)]
            pos_conf = allscal_v[pl.ds(i * 80 + 48, 16)]
            skipf = allscal_v[pl.ds(i * 80 + 64, 16)]
            negc = gath_v[pl.ds(i * 16, 16)]
            skip = skipf > 0.5
            conf_i = (1.5 * pos_conf / jnp.maximum(np_f, 1.0)
                      + negc / jnp.maximum(kf, 1.0))
            total_conf = total_conf + jnp.where(skip, zf + 5.0, conf_i)
            total_loc = total_loc + jnp.where(skip, zf, loc_i)
            num_pos = num_pos + jnp.where(skip, zf, np_f)
        num_pos = jnp.maximum(num_pos, 1.0)
        res = (total_loc / num_pos * _LOC_W
               + total_conf * (_CONF_W / _B))
        out_stage[...] = res
        pltpu.sync_copy(out_stage, out_hbm)


def kernel(bbox_pred, conf_pred, anchors, gt_boxes):
    px1 = bbox_pred[:, :, 0].reshape(_B, _R, _C)
    py1 = bbox_pred[:, :, 1].reshape(_B, _R, _C)
    px2 = bbox_pred[:, :, 2].reshape(_B, _R, _C)
    py2 = bbox_pred[:, :, 3].reshape(_B, _R, _C)
    conf = conf_pred.reshape(_B, _R, _C)
    ax1 = anchors[:, 0].reshape(_R, _C)
    ay1 = anchors[:, 1].reshape(_R, _C)
    ax2 = anchors[:, 2].reshape(_R, _C)
    ay2 = anchors[:, 3].reshape(_R, _C)

    nbits3, scal = pl.pallas_call(
        _dense_body,
        out_shape=(jax.ShapeDtypeStruct((_B, _R, _C), jnp.int32),
                   jax.ShapeDtypeStruct((_B * 5, 16), jnp.float32)),
        in_specs=[pl.BlockSpec(memory_space=pltpu.VMEM)] * 9
        + [pl.BlockSpec(memory_space=pltpu.SMEM)],
        out_specs=(pl.BlockSpec(memory_space=pltpu.VMEM),
                   pl.BlockSpec(memory_space=pltpu.SMEM)),
    )(px1, px2, py1, py2, conf, ax1, ax2, ay1, ay2, gt_boxes)

    nbits = nbits3.reshape(_B * _A)
    res = _sc_mine(nbits, scal.reshape(_B * 5 * 16))
    return res[0]


# speedup vs baseline: 1.0106x; 1.0106x over previous
import functools

import jax
import jax.numpy as jnp
from jax import lax
from jax.experimental import pallas as pl
from jax.experimental.pallas import tpu as pltpu
from jax.experimental.pallas import tpu_sc as plsc

_B, _A, _G = 8, 16384, 24
_R = 128
_C = 128
_IOU_TH, _NEG_POS_RATIO, _CONF_W, _LOC_W, _BETA = 0.5, 3, 2.0, 1.0, 0.05
_NV = _A // 16
_UNROLL = 8


def _smooth_l1(d):
    ad = jnp.abs(d)
    return jnp.where(ad < _BETA, 0.5 * ad * ad / _BETA, ad - 0.5 * _BETA)


def _dense_body(px1, px2, py1, py2, conf, ax1, ax2, ay1, ay2, gt_ref,
                nbits_ref, scal_ref):
    iota = (lax.broadcasted_iota(jnp.int32, (_R, _C), 0) * _C
            + lax.broadcasted_iota(jnp.int32, (_R, _C), 1))
    a1 = ax1[...]
    a2 = ax2[...]
    b1 = ay1[...]
    b2 = ay2[...]
    area_a = (a2 - a1) * (b2 - b1)

    for i in range(_B):
        x1 = px1[i]
        x2 = px2[i]
        y1 = py1[i]
        y2 = py2[i]
        p = conf[i]
        bsum = jnp.sum(x1) + jnp.sum(x2) + jnp.sum(y1) + jnp.sum(y2)
        skip = (bsum == 0.0) | (jnp.max(p) < 0.01)

        def g_body(g, carry):
            best, bestg, force = carry
            gx1 = gt_ref[i, g, 0]
            gy1 = gt_ref[i, g, 1]
            gx2 = gt_ref[i, g, 2]
            gy2 = gt_ref[i, g, 3]
            whx = jnp.maximum(jnp.minimum(a2, gx2) - jnp.maximum(a1, gx1), 0.0)
            why = jnp.maximum(jnp.minimum(b2, gy2) - jnp.maximum(b1, gy1), 0.0)
            inter = whx * why
            area_b = (gx2 - gx1) * (gy2 - gy1)
            union = area_a + area_b - inter
            iou = inter / jnp.maximum(union, 1e-9)
            upd = iou > best
            best = jnp.where(upd, iou, best)
            bestg = jnp.where(upd, g.astype(jnp.float32), bestg)
            m = jnp.max(iou)
            aidx = jnp.min(jnp.where(iou == m, iota, jnp.int32(_A)))
            force = jnp.maximum(force, jnp.where(iota == aidx, 1.0, 0.0))
            return best, bestg, force

        zero = jnp.zeros((_R, _C), jnp.float32)
        best, bestg, force = lax.fori_loop(
            0, _G, g_body,
            (jnp.full((_R, _C), -1.0, jnp.float32), zero, zero))

        mcx = zero
        mcy = zero
        mw = zero
        mh = zero
        for g in range(_G):
            gx1 = gt_ref[i, g, 0]
            gy1 = gt_ref[i, g, 1]
            gx2 = gt_ref[i, g, 2]
            gy2 = gt_ref[i, g, 3]
            sel = bestg == jnp.float32(g)
            mcx = jnp.where(sel, (gx1 + gx2) * 0.5, mcx)
            mcy = jnp.where(sel, (gy1 + gy2) * 0.5, mcy)
            mw = jnp.where(sel, gx2 - gx1, mw)
            mh = jnp.where(sel, gy2 - gy1, mh)

        pos = (best > _IOU_TH) | (force > 0.0)
        posf = pos.astype(jnp.float32)
        np_f = jnp.sum(posf)
        np_i = np_f.astype(jnp.int32)

        ll = (_smooth_l1((x1 + x2) * 0.5 - mcx)
              + _smooth_l1((y1 + y2) * 0.5 - mcy)
              + _smooth_l1((x2 - x1) - mw)
              + _smooth_l1((y2 - y1) - mh))
        loc_i = jnp.sum(ll * posf)

        logp = jnp.maximum(jnp.log(p), -100.0)
        log1mp = jnp.maximum(jnp.log(1.0 - p), -100.0)
        pos_conf = jnp.sum(posf * (-logp))
        neg = jnp.abs(jnp.where(pos, 0.0, -log1mp))
        k = jnp.minimum(np_i * _NEG_POS_RATIO, _A - np_i)

        nbits_ref[i] = lax.bitcast_convert_type(neg, jnp.int32)
        vals = (np_f, k.astype(jnp.float32), loc_i, pos_conf,
                jnp.where(skip, jnp.float32(1.0), jnp.float32(0.0)))
        for f in range(5):
            for j in range(16):
                scal_ref[i * 5 + f, j] = vals[f]


_GD = lax.GatherDimensionNumbers(
    offset_dims=(), collapsed_slice_dims=(0,), start_index_map=(0,))


def _shuf(x, perm):
    return lax.gather(x, perm[:, None], _GD, (1,),
                      mode=lax.GatherScatterMode.PROMISE_IN_BOUNDS)


def _splat_sum(x, iota16):
    for sh in (8, 4, 2, 1):
        x = x + _shuf(x, iota16 ^ sh)
    return x


def _sc_mine_body(nbits_hbm, scal_hbm, out_hbm, row_v, scal5_v, stage_v,
                  gath_v, allscal_v, out_stage, shared):
    c = lax.axis_index("c")
    s = lax.axis_index("s")
    is_worker = (c == 0) & (s < _B)
    iota16 = lax.iota(jnp.int32, 16)
    zf = jnp.zeros((16,), jnp.float32)
    zi = jnp.zeros((16,), jnp.int32)

    @pl.when(is_worker)
    def _worker():
        pltpu.sync_copy(nbits_hbm.at[pl.ds(s * _A, _A)], row_v)
        pltpu.sync_copy(scal_hbm.at[pl.ds(s * 80, 80)], scal5_v)
        kf = scal5_v[pl.ds(16, 16)]
        ki = kf.astype(jnp.int32)

        def bit_body(j, t):
            cand = t | (zi + (jnp.int32(1) << (30 - j)))

            def cnt_body(jj, acc):
                for u in range(_UNROLL):
                    v = row_v[pl.ds(jj * (16 * _UNROLL) + u * 16, 16)]
                    acc = acc + jnp.where(v >= cand, 1, 0).astype(jnp.int32)
                return acc

            cntl = lax.fori_loop(0, _NV // _UNROLL, cnt_body, zi)
            cnt = _splat_sum(cntl, iota16)
            return jnp.where(cnt >= ki, cand, t)

        t_bits = lax.fori_loop(0, 31, bit_body, zi)
        t_val = lax.bitcast_convert_type(t_bits, jnp.float32)

        def sum_body(jj, carry):
            sg, cg = carry
            for u in range(_UNROLL):
                v = row_v[pl.ds(jj * (16 * _UNROLL) + u * 16, 16)]
                m = v > t_bits
                f = lax.bitcast_convert_type(v, jnp.float32)
                sg = sg + jnp.where(m, f, 0.0)
                cg = cg + jnp.where(m, 1.0, 0.0)
            return sg, cg

        sgl, cgl = lax.fori_loop(0, _NV // _UNROLL, sum_body, (zf, zf))
        sum_gt = _splat_sum(sgl, iota16)
        cnt_gt = _splat_sum(cgl, iota16)
        neg_conf = jnp.where(ki > 0, sum_gt + (kf - cnt_gt) * t_val, zf)
        stage_v[...] = neg_conf
        pltpu.sync_copy(stage_v, shared.at[pl.ds(s * 16, 16)])

    plsc.subcore_barrier()

    @pl.when((c == 0) & (s == 0))
    def _leader():
        pltpu.sync_copy(shared, gath_v)
        pltpu.sync_copy(scal_hbm, allscal_v)
        total_loc = zf
        total_conf = zf
        num_pos = zf
        for i in range(_B):
            np_f = allscal_v[pl.ds(i * 80, 16)]
            kf = allscal_v[pl.ds(i * 80 + 16, 16)]
            loc_i = allscal_v[pl.ds(i * 80 + 32, 16)]
            pos_conf = allscal_v[pl.ds(i * 80 + 48, 16)]
            skipf = allscal_v[pl.ds(i * 80 + 64, 16)]
            negc = gath_v[pl.ds(i * 16, 16)]
            skip = skipf > 0.5
            conf_i = (1.5 * pos_conf / jnp.maximum(np_f, 1.0)
                      + negc / jnp.maximum(kf, 1.0))
            total_conf = total_conf + jnp.where(skip, zf + 5.0, conf_i)
            total_loc = total_loc + jnp.where(skip, zf, loc_i)
            num_pos = num_pos + jnp.where(skip, zf, np_f)
        num_pos = jnp.maximum(num_pos, 1.0)
        res = (total_loc / num_pos * _LOC_W
               + total_conf * (_CONF_W / _B))
        out_stage[...] = res
        pltpu.sync_copy(out_stage, out_hbm)


@functools.cache
def _get_sc_mine():
    mesh = plsc.VectorSubcoreMesh(core_axis_name="c", subcore_axis_name="s")
    return pl.kernel(
        _sc_mine_body,
        mesh=mesh,
        compiler_params=pltpu.CompilerParams(needs_layout_passes=False),
        out_type=jax.ShapeDtypeStruct((16,), jnp.float32),
        scratch_types=[
            pltpu.VMEM((_A,), jnp.int32),
            pltpu.VMEM((80,), jnp.float32),
            pltpu.VMEM((16,), jnp.float32),
            pltpu.VMEM((_B * 16,), jnp.float32),
            pltpu.VMEM((_B * 80,), jnp.float32),
            pltpu.VMEM((16,), jnp.float32),
            pltpu.VMEM_SHARED((_B * 16,), jnp.float32),
        ],
    )


def kernel(bbox_pred, conf_pred, anchors, gt_boxes):
    px1 = bbox_pred[:, :, 0].reshape(_B, _R, _C)
    py1 = bbox_pred[:, :, 1].reshape(_B, _R, _C)
    px2 = bbox_pred[:, :, 2].reshape(_B, _R, _C)
    py2 = bbox_pred[:, :, 3].reshape(_B, _R, _C)
    conf = conf_pred.reshape(_B, _R, _C)
    ax1 = anchors[:, 0].reshape(_R, _C)
    ay1 = anchors[:, 1].reshape(_R, _C)
    ax2 = anchors[:, 2].reshape(_R, _C)
    ay2 = anchors[:, 3].reshape(_R, _C)

    nbits3, scal = pl.pallas_call(
        _dense_body,
        out_shape=(jax.ShapeDtypeStruct((_B, _R, _C), jnp.int32),
                   jax.ShapeDtypeStruct((_B * 5, 16), jnp.float32)),
        in_specs=[pl.BlockSpec(memory_space=pltpu.VMEM)] * 9
        + [pl.BlockSpec(memory_space=pltpu.SMEM)],
        out_specs=(pl.BlockSpec(memory_space=pltpu.VMEM),
                   pl.BlockSpec(memory_space=pltpu.SMEM)),
    )(px1, px2, py1, py2, conf, ax1, ax2, ay1, ay2, gt_boxes)

    nbits = nbits3.reshape(_B * _A)
    res = _get_sc_mine()(nbits, scal.reshape(_B * 5 * 16))
    return res[0]


# speedup vs baseline: 1.1406x; 1.1287x over previous
import functools

import jax
import jax.numpy as jnp
from jax import lax
from jax.experimental import pallas as pl
from jax.experimental.pallas import tpu as pltpu
from jax.experimental.pallas import tpu_sc as plsc

_B, _A, _G = 8, 16384, 24
_R = 128
_C = 128
_IOU_TH, _NEG_POS_RATIO, _CONF_W, _LOC_W, _BETA = 0.5, 3, 2.0, 1.0, 0.05
_NV = _A // 16
_UNROLL = 8


def _smooth_l1(d):
    ad = jnp.abs(d)
    return jnp.where(ad < _BETA, 0.5 * ad * ad / _BETA, ad - 0.5 * _BETA)


def _dense_body(px1, px2, py1, py2, conf, ax1, ax2, ay1, ay2, gt_ref,
                nbits_ref, scal_ref):
    niota = -(lax.broadcasted_iota(jnp.int32, (_R, _C), 0) * _C
              + lax.broadcasted_iota(jnp.int32, (_R, _C), 1)).astype(jnp.float32)
    a1 = ax1[...]
    a2 = ax2[...]
    b1 = ay1[...]
    b2 = ay2[...]
    area_a = (a2 - a1) * (b2 - b1)

    for i in range(_B):
        x1 = px1[i]
        x2 = px2[i]
        y1 = py1[i]
        y2 = py2[i]
        p = conf[i]
        bsum = jnp.sum(x1) + jnp.sum(x2) + jnp.sum(y1) + jnp.sum(y2)
        skip = (bsum == 0.0) | (jnp.max(p) < 0.01)

        def g_body(g, carry):
            best, bestg, force = carry
            gx1 = gt_ref[i, g, 0]
            gy1 = gt_ref[i, g, 1]
            gx2 = gt_ref[i, g, 2]
            gy2 = gt_ref[i, g, 3]
            whx = jnp.maximum(jnp.minimum(a2, gx2) - jnp.maximum(a1, gx1), 0.0)
            why = jnp.maximum(jnp.minimum(b2, gy2) - jnp.maximum(b1, gy1), 0.0)
            inter = whx * why
            area_b = (gx2 - gx1) * (gy2 - gy1)
            union = area_a + area_b - inter
            iou = inter / jnp.maximum(union, 1e-9)
            upd = iou > best
            best = jnp.where(upd, iou, best)
            bestg = jnp.where(upd, g.astype(jnp.float32), bestg)
            m = jnp.max(iou)
            aidxn = jnp.max(jnp.where(iou == m, niota, jnp.float32(-1e9)))
            force = jnp.maximum(force, jnp.where(niota == aidxn, 1.0, 0.0))
            return best, bestg, force

        zero = jnp.zeros((_R, _C), jnp.float32)
        best, bestg, force = lax.fori_loop(
            0, _G, g_body,
            (jnp.full((_R, _C), -1.0, jnp.float32), zero, zero))

        mcx = zero
        mcy = zero
        mw = zero
        mh = zero
        for g in range(_G):
            gx1 = gt_ref[i, g, 0]
            gy1 = gt_ref[i, g, 1]
            gx2 = gt_ref[i, g, 2]
            gy2 = gt_ref[i, g, 3]
            sel = bestg == jnp.float32(g)
            mcx = jnp.where(sel, (gx1 + gx2) * 0.5, mcx)
            mcy = jnp.where(sel, (gy1 + gy2) * 0.5, mcy)
            mw = jnp.where(sel, gx2 - gx1, mw)
            mh = jnp.where(sel, gy2 - gy1, mh)

        pos = (best > _IOU_TH) | (force > 0.0)
        posf = pos.astype(jnp.float32)
        np_f = jnp.sum(posf)
        np_i = np_f.astype(jnp.int32)

        ll = (_smooth_l1((x1 + x2) * 0.5 - mcx)
              + _smooth_l1((y1 + y2) * 0.5 - mcy)
              + _smooth_l1((x2 - x1) - mw)
              + _smooth_l1((y2 - y1) - mh))
        loc_i = jnp.sum(ll * posf)

        logp = jnp.maximum(jnp.log(p), -100.0)
        log1mp = jnp.maximum(jnp.log(1.0 - p), -100.0)
        pos_conf = jnp.sum(posf * (-logp))
        neg = jnp.abs(jnp.where(pos, 0.0, -log1mp))
        k = jnp.minimum(np_i * _NEG_POS_RATIO, _A - np_i)

        nbits_ref[i] = lax.bitcast_convert_type(neg, jnp.int32)
        vals = (np_f, k.astype(jnp.float32), loc_i, pos_conf,
                jnp.where(skip, jnp.float32(1.0), jnp.float32(0.0)))
        for f in range(5):
            for j in range(16):
                scal_ref[i * 5 + f, j] = vals[f]


_GD = lax.GatherDimensionNumbers(
    offset_dims=(), collapsed_slice_dims=(0,), start_index_map=(0,))


def _shuf(x, perm):
    return lax.gather(x, perm[:, None], _GD, (1,),
                      mode=lax.GatherScatterMode.PROMISE_IN_BOUNDS)


def _splat_sum(x, iota16):
    for sh in (8, 4, 2, 1):
        x = x + _shuf(x, iota16 ^ sh)
    return x


def _sc_mine_body(nbits_hbm, scal_hbm, out_hbm, row_v, scal5_v, stage_v,
                  gath_v, allscal_v, out_stage, shared):
    c = lax.axis_index("c")
    s = lax.axis_index("s")
    is_worker = (c == 0) & (s < _B)
    iota16 = lax.iota(jnp.int32, 16)
    zf = jnp.zeros((16,), jnp.float32)
    zi = jnp.zeros((16,), jnp.int32)

    @pl.when(is_worker)
    def _worker():
        pltpu.sync_copy(nbits_hbm.at[pl.ds(s * _A, _A)], row_v)
        pltpu.sync_copy(scal_hbm.at[pl.ds(s * 80, 80)], scal5_v)
        kf = scal5_v[pl.ds(16, 16)]
        ki = kf.astype(jnp.int32)

        def bit_body(j, t):
            cand = t | (zi + (jnp.int32(1) << (30 - j)))

            def cnt_body(jj, acc):
                for u in range(_UNROLL):
                    v = row_v[pl.ds(jj * (16 * _UNROLL) + u * 16, 16)]
                    acc = acc + jnp.where(v >= cand, 1, 0).astype(jnp.int32)
                return acc

            cntl = lax.fori_loop(0, _NV // _UNROLL, cnt_body, zi)
            cnt = _splat_sum(cntl, iota16)
            return jnp.where(cnt >= ki, cand, t)

        t_bits = lax.fori_loop(0, 31, bit_body, zi)
        t_val = lax.bitcast_convert_type(t_bits, jnp.float32)

        def sum_body(jj, carry):
            sg, cg = carry
            for u in range(_UNROLL):
                v = row_v[pl.ds(jj * (16 * _UNROLL) + u * 16, 16)]
                m = v > t_bits
                f = lax.bitcast_convert_type(v, jnp.float32)
                sg = sg + jnp.where(m, f, 0.0)
                cg = cg + jnp.where(m, 1.0, 0.0)
            return sg, cg

        sgl, cgl = lax.fori_loop(0, _NV // _UNROLL, sum_body, (zf, zf))
        sum_gt = _splat_sum(sgl, iota16)
        cnt_gt = _splat_sum(cgl, iota16)
        neg_conf = jnp.where(ki > 0, sum_gt + (kf - cnt_gt) * t_val, zf)
        stage_v[...] = neg_conf
        pltpu.sync_copy(stage_v, shared.at[pl.ds(s * 16, 16)])

    plsc.subcore_barrier()

    @pl.when((c == 0) & (s == 0))
    def _leader():
        pltpu.sync_copy(shared, gath_v)
        pltpu.sync_copy(scal_hbm, allscal_v)
        total_loc = zf
        total_conf = zf
        num_pos = zf
        for i in range(_B):
            np_f = allscal_v[pl.ds(i * 80, 16)]
            kf = allscal_v[pl.ds(i * 80 + 16, 16)]
            loc_i = allscal_v[pl.ds(i * 80 + 32, 16)]
            pos_conf = allscal_v[pl.ds(i * 80 + 48, 16)]
            skipf = allscal_v[pl.ds(i * 80 + 64, 16)]
            negc = gath_v[pl.ds(i * 16, 16)]
            skip = skipf > 0.5
            conf_i = (1.5 * pos_conf / jnp.maximum(np_f, 1.0)
                      + negc / jnp.maximum(kf, 1.0))
            total_conf = total_conf + jnp.where(skip, zf + 5.0, conf_i)
            total_loc = total_loc + jnp.where(skip, zf, loc_i)
            num_pos = num_pos + jnp.where(skip, zf, np_f)
        num_pos = jnp.maximum(num_pos, 1.0)
        res = (total_loc / num_pos * _LOC_W
               + total_conf * (_CONF_W / _B))
        out_stage[...] = res
        pltpu.sync_copy(out_stage, out_hbm)


@functools.cache
def _get_sc_mine():
    mesh = plsc.VectorSubcoreMesh(core_axis_name="c", subcore_axis_name="s")
    return pl.kernel(
        _sc_mine_body,
        mesh=mesh,
        compiler_params=pltpu.CompilerParams(needs_layout_passes=False),
        out_type=jax.ShapeDtypeStruct((16,), jnp.float32),
        scratch_types=[
            pltpu.VMEM((_A,), jnp.int32),
            pltpu.VMEM((80,), jnp.float32),
            pltpu.VMEM((16,), jnp.float32),
            pltpu.VMEM((_B * 16,), jnp.float32),
            pltpu.VMEM((_B * 80,), jnp.float32),
            pltpu.VMEM((16,), jnp.float32),
            pltpu.VMEM_SHARED((_B * 16,), jnp.float32),
        ],
    )


def kernel(bbox_pred, conf_pred, anchors, gt_boxes):
    px1 = bbox_pred[:, :, 0].reshape(_B, _R, _C)
    py1 = bbox_pred[:, :, 1].reshape(_B, _R, _C)
    px2 = bbox_pred[:, :, 2].reshape(_B, _R, _C)
    py2 = bbox_pred[:, :, 3].reshape(_B, _R, _C)
    conf = conf_pred.reshape(_B, _R, _C)
    ax1 = anchors[:, 0].reshape(_R, _C)
    ay1 = anchors[:, 1].reshape(_R, _C)
    ax2 = anchors[:, 2].reshape(_R, _C)
    ay2 = anchors[:, 3].reshape(_R, _C)

    nbits3, scal = pl.pallas_call(
        _dense_body,
        out_shape=(jax.ShapeDtypeStruct((_B, _R, _C), jnp.int32),
                   jax.ShapeDtypeStruct((_B * 5, 16), jnp.float32)),
        in_specs=[pl.BlockSpec(memory_space=pltpu.VMEM)] * 9
        + [pl.BlockSpec(memory_space=pltpu.SMEM)],
        out_specs=(pl.BlockSpec(memory_space=pltpu.VMEM),
                   pl.BlockSpec(memory_space=pltpu.SMEM)),
    )(px1, px2, py1, py2, conf, ax1, ax2, ay1, ay2, gt_boxes)

    nbits = nbits3.reshape(_B * _A)
    res = _get_sc_mine()(nbits, scal.reshape(_B * 5 * 16))
    return res[0]


# speedup vs baseline: 1.4786x; 1.2963x over previous
import functools

import jax
import jax.numpy as jnp
from jax import lax
from jax.experimental import pallas as pl
from jax.experimental.pallas import tpu as pltpu
from jax.experimental.pallas import tpu_sc as plsc

_B, _A, _G = 8, 16384, 24
_R = 128
_C = 128
_IOU_TH, _NEG_POS_RATIO, _CONF_W, _LOC_W, _BETA = 0.5, 3, 2.0, 1.0, 0.05
_NV = _A // 16
_UNROLL = 8


def _smooth_l1(d):
    ad = jnp.abs(d)
    return jnp.where(ad < _BETA, 0.5 * ad * ad / _BETA, ad - 0.5 * _BETA)


def _dense_body(px1, px2, py1, py2, conf, ax1, ax2, ay1, ay2, gt_ref,
                nbits_ref, scal_ref):
    niota = -(lax.broadcasted_iota(jnp.int32, (_R, _C), 0) * _C
              + lax.broadcasted_iota(jnp.int32, (_R, _C), 1)).astype(jnp.float32)
    a1 = ax1[...]
    a2 = ax2[...]
    b1 = ay1[...]
    b2 = ay2[...]
    area_a = (a2 - a1) * (b2 - b1)

    for i in range(_B):
        x1 = px1[i]
        x2 = px2[i]
        y1 = py1[i]
        y2 = py2[i]
        p = conf[i]
        bsum = jnp.sum(x1) + jnp.sum(x2) + jnp.sum(y1) + jnp.sum(y2)
        skip = (bsum == 0.0) | (jnp.max(p) < 0.01)

        def g_body(g, carry):
            best, bestg, force = carry
            gx1 = gt_ref[i, g, 0]
            gy1 = gt_ref[i, g, 1]
            gx2 = gt_ref[i, g, 2]
            gy2 = gt_ref[i, g, 3]
            whx = jnp.maximum(jnp.minimum(a2, gx2) - jnp.maximum(a1, gx1), 0.0)
            why = jnp.maximum(jnp.minimum(b2, gy2) - jnp.maximum(b1, gy1), 0.0)
            inter = whx * why
            area_b = (gx2 - gx1) * (gy2 - gy1)
            union = area_a + area_b - inter
            iou = inter / jnp.maximum(union, 1e-9)
            upd = iou > best
            best = jnp.where(upd, iou, best)
            bestg = jnp.where(upd, g.astype(jnp.float32), bestg)
            m = jnp.max(iou)
            aidxn = jnp.max(jnp.where(iou == m, niota, jnp.float32(-1e9)))
            force = jnp.maximum(force, jnp.where(niota == aidxn, 1.0, 0.0))
            return best, bestg, force

        zero = jnp.zeros((_R, _C), jnp.float32)
        best, bestg, force = lax.fori_loop(
            0, _G, g_body,
            (jnp.full((_R, _C), -1.0, jnp.float32), zero, zero),
            unroll=6)

        mcx = zero
        mcy = zero
        mw = zero
        mh = zero
        for g in range(_G):
            gx1 = gt_ref[i, g, 0]
            gy1 = gt_ref[i, g, 1]
            gx2 = gt_ref[i, g, 2]
            gy2 = gt_ref[i, g, 3]
            sel = bestg == jnp.float32(g)
            mcx = jnp.where(sel, (gx1 + gx2) * 0.5, mcx)
            mcy = jnp.where(sel, (gy1 + gy2) * 0.5, mcy)
            mw = jnp.where(sel, gx2 - gx1, mw)
            mh = jnp.where(sel, gy2 - gy1, mh)

        pos = (best > _IOU_TH) | (force > 0.0)
        posf = pos.astype(jnp.float32)
        np_f = jnp.sum(posf)
        np_i = np_f.astype(jnp.int32)

        ll = (_smooth_l1((x1 + x2) * 0.5 - mcx)
              + _smooth_l1((y1 + y2) * 0.5 - mcy)
              + _smooth_l1((x2 - x1) - mw)
              + _smooth_l1((y2 - y1) - mh))
        loc_i = jnp.sum(ll * posf)

        logp = jnp.maximum(jnp.log(p), -100.0)
        log1mp = jnp.maximum(jnp.log(1.0 - p), -100.0)
        pos_conf = jnp.sum(posf * (-logp))
        neg = jnp.abs(jnp.where(pos, 0.0, -log1mp))
        k = jnp.minimum(np_i * _NEG_POS_RATIO, _A - np_i)

        nbits_ref[i] = lax.bitcast_convert_type(neg, jnp.int32)
        vals = (np_f, k.astype(jnp.float32), loc_i, pos_conf,
                jnp.where(skip, jnp.float32(1.0), jnp.float32(0.0)))
        for f in range(5):
            for j in range(16):
                scal_ref[i * 5 + f, j] = vals[f]


_GD = lax.GatherDimensionNumbers(
    offset_dims=(), collapsed_slice_dims=(0,), start_index_map=(0,))


def _shuf(x, perm):
    return lax.gather(x, perm[:, None], _GD, (1,),
                      mode=lax.GatherScatterMode.PROMISE_IN_BOUNDS)


def _splat_sum(x, iota16):
    for sh in (8, 4, 2, 1):
        x = x + _shuf(x, iota16 ^ sh)
    return x


def _sc_mine_body(nbits_hbm, scal_hbm, out_hbm, row_v, scal5_v, stage_v,
                  gath_v, allscal_v, out_stage, shared):
    c = lax.axis_index("c")
    s = lax.axis_index("s")
    is_worker = (c == 0) & (s < _B)
    iota16 = lax.iota(jnp.int32, 16)
    zf = jnp.zeros((16,), jnp.float32)
    zi = jnp.zeros((16,), jnp.int32)

    @pl.when(is_worker)
    def _worker():
        pltpu.sync_copy(nbits_hbm.at[pl.ds(s * _A, _A)], row_v)
        pltpu.sync_copy(scal_hbm.at[pl.ds(s * 80, 80)], scal5_v)
        kf = scal5_v[pl.ds(16, 16)]
        ki = kf.astype(jnp.int32)

        def bit_body(j, t):
            cand = t | (zi + (jnp.int32(1) << (30 - j)))

            def cnt_body(jj, acc):
                for u in range(_UNROLL):
                    v = row_v[pl.ds(jj * (16 * _UNROLL) + u * 16, 16)]
                    acc = acc + jnp.where(v >= cand, 1, 0).astype(jnp.int32)
                return acc

            cntl = lax.fori_loop(0, _NV // _UNROLL, cnt_body, zi)
            cnt = _splat_sum(cntl, iota16)
            return jnp.where(cnt >= ki, cand, t)

        t_bits = lax.fori_loop(0, 31, bit_body, zi)
        t_val = lax.bitcast_convert_type(t_bits, jnp.float32)

        def sum_body(jj, carry):
            sg, cg = carry
            for u in range(_UNROLL):
                v = row_v[pl.ds(jj * (16 * _UNROLL) + u * 16, 16)]
                m = v > t_bits
                f = lax.bitcast_convert_type(v, jnp.float32)
                sg = sg + jnp.where(m, f, 0.0)
                cg = cg + jnp.where(m, 1.0, 0.0)
            return sg, cg

        sgl, cgl = lax.fori_loop(0, _NV // _UNROLL, sum_body, (zf, zf))
        sum_gt = _splat_sum(sgl, iota16)
        cnt_gt = _splat_sum(cgl, iota16)
        neg_conf = jnp.where(ki > 0, sum_gt + (kf - cnt_gt) * t_val, zf)
        stage_v[...] = neg_conf
        pltpu.sync_copy(stage_v, shared.at[pl.ds(s * 16, 16)])

    plsc.subcore_barrier()

    @pl.when((c == 0) & (s == 0))
    def _leader():
        pltpu.sync_copy(shared, gath_v)
        pltpu.sync_copy(scal_hbm, allscal_v)
        total_loc = zf
        total_conf = zf
        num_pos = zf
        for i in range(_B):
            np_f = allscal_v[pl.ds(i * 80, 16)]
            kf = allscal_v[pl.ds(i * 80 + 16, 16)]
            loc_i = allscal_v[pl.ds(i * 80 + 32, 16)]
            pos_conf = allscal_v[pl.ds(i * 80 + 48, 16)]
            skipf = allscal_v[pl.ds(i * 80 + 64, 16)]
            negc = gath_v[pl.ds(i * 16, 16)]
            skip = skipf > 0.5
            conf_i = (1.5 * pos_conf / jnp.maximum(np_f, 1.0)
                      + negc / jnp.maximum(kf, 1.0))
            total_conf = total_conf + jnp.where(skip, zf + 5.0, conf_i)
            total_loc = total_loc + jnp.where(skip, zf, loc_i)
            num_pos = num_pos + jnp.where(skip, zf, np_f)
        num_pos = jnp.maximum(num_pos, 1.0)
        res = (total_loc / num_pos * _LOC_W
               + total_conf * (_CONF_W / _B))
        out_stage[...] = res
        pltpu.sync_copy(out_stage, out_hbm)


@functools.cache
def _get_sc_mine():
    mesh = plsc.VectorSubcoreMesh(core_axis_name="c", subcore_axis_name="s")
    return pl.kernel(
        _sc_mine_body,
        mesh=mesh,
        compiler_params=pltpu.CompilerParams(needs_layout_passes=False),
        out_type=jax.ShapeDtypeStruct((16,), jnp.float32),
        scratch_types=[
            pltpu.VMEM((_A,), jnp.int32),
            pltpu.VMEM((80,), jnp.float32),
            pltpu.VMEM((16,), jnp.float32),
            pltpu.VMEM((_B * 16,), jnp.float32),
            pltpu.VMEM((_B * 80,), jnp.float32),
            pltpu.VMEM((16,), jnp.float32),
            pltpu.VMEM_SHARED((_B * 16,), jnp.float32),
        ],
    )


def kernel(bbox_pred, conf_pred, anchors, gt_boxes):
    px1 = bbox_pred[:, :, 0].reshape(_B, _R, _C)
    py1 = bbox_pred[:, :, 1].reshape(_B, _R, _C)
    px2 = bbox_pred[:, :, 2].reshape(_B, _R, _C)
    py2 = bbox_pred[:, :, 3].reshape(_B, _R, _C)
    conf = conf_pred.reshape(_B, _R, _C)
    ax1 = anchors[:, 0].reshape(_R, _C)
    ay1 = anchors[:, 1].reshape(_R, _C)
    ax2 = anchors[:, 2].reshape(_R, _C)
    ay2 = anchors[:, 3].reshape(_R, _C)

    nbits3, scal = pl.pallas_call(
        _dense_body,
        out_shape=(jax.ShapeDtypeStruct((_B, _R, _C), jnp.int32),
                   jax.ShapeDtypeStruct((_B * 5, 16), jnp.float32)),
        in_specs=[pl.BlockSpec(memory_space=pltpu.VMEM)] * 9
        + [pl.BlockSpec(memory_space=pltpu.SMEM)],
        out_specs=(pl.BlockSpec(memory_space=pltpu.VMEM),
                   pl.BlockSpec(memory_space=pltpu.SMEM)),
    )(px1, px2, py1, py2, conf, ax1, ax2, ay1, ay2, gt_boxes)

    nbits = nbits3.reshape(_B * _A)
    res = _get_sc_mine()(nbits, scal.reshape(_B * 5 * 16))
    return res[0]


# speedup vs baseline: 1.5594x; 1.0547x over previous
import functools

import jax
import jax.numpy as jnp
from jax import lax
from jax.experimental import pallas as pl
from jax.experimental.pallas import tpu as pltpu
from jax.experimental.pallas import tpu_sc as plsc

_B, _A, _G = 8, 16384, 24
_R = 128
_C = 128
_IOU_TH, _NEG_POS_RATIO, _CONF_W, _LOC_W, _BETA = 0.5, 3, 2.0, 1.0, 0.05
_NV = _A // 16
_UNROLL = 8


def _smooth_l1(d):
    ad = jnp.abs(d)
    return jnp.where(ad < _BETA, 0.5 * ad * ad / _BETA, ad - 0.5 * _BETA)


def _dense_body(px1, px2, py1, py2, conf, ax1, ax2, ay1, ay2, gt_ref,
                nbits_ref, scal_ref):
    niota = -(lax.broadcasted_iota(jnp.int32, (_R, _C), 0) * _C
              + lax.broadcasted_iota(jnp.int32, (_R, _C), 1)).astype(jnp.float32)
    a1 = ax1[...]
    a2 = ax2[...]
    b1 = ay1[...]
    b2 = ay2[...]
    area_a = (a2 - a1) * (b2 - b1)

    for i in range(_B):
        x1 = px1[i]
        x2 = px2[i]
        y1 = py1[i]
        y2 = py2[i]
        p = conf[i]
        bsum = jnp.sum(x1) + jnp.sum(x2) + jnp.sum(y1) + jnp.sum(y2)
        skip = (bsum == 0.0) | (jnp.max(p) < 0.01)

        def g_body(g, carry):
            best, bestg, force = carry
            gx1 = gt_ref[i, g, 0]
            gy1 = gt_ref[i, g, 1]
            gx2 = gt_ref[i, g, 2]
            gy2 = gt_ref[i, g, 3]
            whx = jnp.maximum(jnp.minimum(a2, gx2) - jnp.maximum(a1, gx1), 0.0)
            why = jnp.maximum(jnp.minimum(b2, gy2) - jnp.maximum(b1, gy1), 0.0)
            inter = whx * why
            area_b = (gx2 - gx1) * (gy2 - gy1)
            union = area_a + area_b - inter
            iou = inter / jnp.maximum(union, 1e-9)
            upd = iou > best
            best = jnp.where(upd, iou, best)
            bestg = jnp.where(upd, g.astype(jnp.float32), bestg)
            m = jnp.max(iou)
            aidxn = jnp.max(jnp.where(iou == m, niota, jnp.float32(-1e9)))
            force = jnp.maximum(force, jnp.where(niota == aidxn, 1.0, 0.0))
            return best, bestg, force

        zero = jnp.zeros((_R, _C), jnp.float32)
        best, bestg, force = lax.fori_loop(
            0, _G, g_body,
            (jnp.full((_R, _C), -1.0, jnp.float32), zero, zero),
            unroll=_G)

        mcx = zero
        mcy = zero
        mw = zero
        mh = zero
        for g in range(_G):
            gx1 = gt_ref[i, g, 0]
            gy1 = gt_ref[i, g, 1]
            gx2 = gt_ref[i, g, 2]
            gy2 = gt_ref[i, g, 3]
            sel = bestg == jnp.float32(g)
            mcx = jnp.where(sel, (gx1 + gx2) * 0.5, mcx)
            mcy = jnp.where(sel, (gy1 + gy2) * 0.5, mcy)
            mw = jnp.where(sel, gx2 - gx1, mw)
            mh = jnp.where(sel, gy2 - gy1, mh)

        pos = (best > _IOU_TH) | (force > 0.0)
        posf = pos.astype(jnp.float32)
        np_f = jnp.sum(posf)
        np_i = np_f.astype(jnp.int32)

        ll = (_smooth_l1((x1 + x2) * 0.5 - mcx)
              + _smooth_l1((y1 + y2) * 0.5 - mcy)
              + _smooth_l1((x2 - x1) - mw)
              + _smooth_l1((y2 - y1) - mh))
        loc_i = jnp.sum(ll * posf)

        logp = jnp.maximum(jnp.log(p), -100.0)
        log1mp = jnp.maximum(jnp.log(1.0 - p), -100.0)
        pos_conf = jnp.sum(posf * (-logp))
        neg = jnp.abs(jnp.where(pos, 0.0, -log1mp))
        k = jnp.minimum(np_i * _NEG_POS_RATIO, _A - np_i)

        nbits_ref[i] = lax.bitcast_convert_type(neg, jnp.int32)
        vals = (np_f, k.astype(jnp.float32), loc_i, pos_conf,
                jnp.where(skip, jnp.float32(1.0), jnp.float32(0.0)))
        for f in range(5):
            for j in range(16):
                scal_ref[i * 5 + f, j] = vals[f]


_GD = lax.GatherDimensionNumbers(
    offset_dims=(), collapsed_slice_dims=(0,), start_index_map=(0,))


def _shuf(x, perm):
    return lax.gather(x, perm[:, None], _GD, (1,),
                      mode=lax.GatherScatterMode.PROMISE_IN_BOUNDS)


def _splat_sum(x, iota16):
    for sh in (8, 4, 2, 1):
        x = x + _shuf(x, iota16 ^ sh)
    return x


def _sc_mine_body(nbits_hbm, scal_hbm, out_hbm, row_v, scal5_v, stage_v,
                  gath_v, allscal_v, out_stage, shared):
    c = lax.axis_index("c")
    s = lax.axis_index("s")
    is_worker = (c == 0) & (s < _B)
    iota16 = lax.iota(jnp.int32, 16)
    zf = jnp.zeros((16,), jnp.float32)
    zi = jnp.zeros((16,), jnp.int32)

    @pl.when(is_worker)
    def _worker():
        pltpu.sync_copy(nbits_hbm.at[pl.ds(s * _A, _A)], row_v)
        pltpu.sync_copy(scal_hbm.at[pl.ds(s * 80, 80)], scal5_v)
        kf = scal5_v[pl.ds(16, 16)]
        ki = kf.astype(jnp.int32)

        def bit_body(j, t):
            cand = t | (zi + (jnp.int32(1) << (30 - j)))

            def cnt_body(jj, acc):
                for u in range(_UNROLL):
                    v = row_v[pl.ds(jj * (16 * _UNROLL) + u * 16, 16)]
                    acc = acc + jnp.where(v >= cand, 1, 0).astype(jnp.int32)
                return acc

            cntl = lax.fori_loop(0, _NV // _UNROLL, cnt_body, zi)
            cnt = _splat_sum(cntl, iota16)
            return jnp.where(cnt >= ki, cand, t)

        t_bits = lax.fori_loop(0, 31, bit_body, zi)
        t_val = lax.bitcast_convert_type(t_bits, jnp.float32)

        def sum_body(jj, carry):
            sg, cg = carry
            for u in range(_UNROLL):
                v = row_v[pl.ds(jj * (16 * _UNROLL) + u * 16, 16)]
                m = v > t_bits
                f = lax.bitcast_convert_type(v, jnp.float32)
                sg = sg + jnp.where(m, f, 0.0)
                cg = cg + jnp.where(m, 1.0, 0.0)
            return sg, cg

        sgl, cgl = lax.fori_loop(0, _NV // _UNROLL, sum_body, (zf, zf))
        sum_gt = _splat_sum(sgl, iota16)
        cnt_gt = _splat_sum(cgl, iota16)
        neg_conf = jnp.where(ki > 0, sum_gt + (kf - cnt_gt) * t_val, zf)
        stage_v[...] = neg_conf
        pltpu.sync_copy(stage_v, shared.at[pl.ds(s * 16, 16)])

    plsc.subcore_barrier()

    @pl.when((c == 0) & (s == 0))
    def _leader():
        pltpu.sync_copy(shared, gath_v)
        pltpu.sync_copy(scal_hbm, allscal_v)
        total_loc = zf
        total_conf = zf
        num_pos = zf
        for i in range(_B):
            np_f = allscal_v[pl.ds(i * 80, 16)]
            kf = allscal_v[pl.ds(i * 80 + 16, 16)]
            loc_i = allscal_v[pl.ds(i * 80 + 32, 16)]
            pos_conf = allscal_v[pl.ds(i * 80 + 48, 16)]
            skipf = allscal_v[pl.ds(i * 80 + 64, 16)]
            negc = gath_v[pl.ds(i * 16, 16)]
            skip = skipf > 0.5
            conf_i = (1.5 * pos_conf / jnp.maximum(np_f, 1.0)
                      + negc / jnp.maximum(kf, 1.0))
            total_conf = total_conf + jnp.where(skip, zf + 5.0, conf_i)
            total_loc = total_loc + jnp.where(skip, zf, loc_i)
            num_pos = num_pos + jnp.where(skip, zf, np_f)
        num_pos = jnp.maximum(num_pos, 1.0)
        res = (total_loc / num_pos * _LOC_W
               + total_conf * (_CONF_W / _B))
        out_stage[...] = res
        pltpu.sync_copy(out_stage, out_hbm)


@functools.cache
def _get_sc_mine():
    mesh = plsc.VectorSubcoreMesh(core_axis_name="c", subcore_axis_name="s")
    return pl.kernel(
        _sc_mine_body,
        mesh=mesh,
        compiler_params=pltpu.CompilerParams(needs_layout_passes=False),
        out_type=jax.ShapeDtypeStruct((16,), jnp.float32),
        scratch_types=[
            pltpu.VMEM((_A,), jnp.int32),
            pltpu.VMEM((80,), jnp.float32),
            pltpu.VMEM((16,), jnp.float32),
            pltpu.VMEM((_B * 16,), jnp.float32),
            pltpu.VMEM((_B * 80,), jnp.float32),
            pltpu.VMEM((16,), jnp.float32),
            pltpu.VMEM_SHARED((_B * 16,), jnp.float32),
        ],
    )


def kernel(bbox_pred, conf_pred, anchors, gt_boxes):
    px1 = bbox_pred[:, :, 0].reshape(_B, _R, _C)
    py1 = bbox_pred[:, :, 1].reshape(_B, _R, _C)
    px2 = bbox_pred[:, :, 2].reshape(_B, _R, _C)
    py2 = bbox_pred[:, :, 3].reshape(_B, _R, _C)
    conf = conf_pred.reshape(_B, _R, _C)
    ax1 = anchors[:, 0].reshape(_R, _C)
    ay1 = anchors[:, 1].reshape(_R, _C)
    ax2 = anchors[:, 2].reshape(_R, _C)
    ay2 = anchors[:, 3].reshape(_R, _C)

    nbits3, scal = pl.pallas_call(
        _dense_body,
        out_shape=(jax.ShapeDtypeStruct((_B, _R, _C), jnp.int32),
                   jax.ShapeDtypeStruct((_B * 5, 16), jnp.float32)),
        in_specs=[pl.BlockSpec(memory_space=pltpu.VMEM)] * 9
        + [pl.BlockSpec(memory_space=pltpu.SMEM)],
        out_specs=(pl.BlockSpec(memory_space=pltpu.VMEM),
                   pl.BlockSpec(memory_space=pltpu.SMEM)),
    )(px1, px2, py1, py2, conf, ax1, ax2, ay1, ay2, gt_boxes)

    nbits = nbits3.reshape(_B * _A)
    res = _get_sc_mine()(nbits, scal.reshape(_B * 5 * 16))
    return res[0]


# speedup vs baseline: 1.5960x; 1.0235x over previous
import functools

import jax
import jax.numpy as jnp
from jax import lax
from jax.experimental import pallas as pl
from jax.experimental.pallas import tpu as pltpu
from jax.experimental.pallas import tpu_sc as plsc

_B, _A, _G = 8, 16384, 24
_R = 128
_C = 128
_IOU_TH, _NEG_POS_RATIO, _CONF_W, _LOC_W, _BETA = 0.5, 3, 2.0, 1.0, 0.05
_NV = _A // 16
_UNROLL = 8


def _smooth_l1(d):
    ad = jnp.abs(d)
    return jnp.where(ad < _BETA, 0.5 * ad * ad / _BETA, ad - 0.5 * _BETA)


_NC = 16
_CR = _R // _NC
_CE = _CR * _C
_GG = 8


def _dense_body(px1, px2, py1, py2, conf, ax1, ax2, ay1, ay2, gt_ref,
                nbits_ref, scal_ref):
    niota = -(lax.broadcasted_iota(jnp.int32, (_R, _C), 0) * _C
              + lax.broadcasted_iota(jnp.int32, (_R, _C), 1)).astype(jnp.float32)
    npos_iota = -(lax.broadcasted_iota(jnp.int32, (_CR, _C), 0) * _C
                  + lax.broadcasted_iota(jnp.int32, (_CR, _C), 1)
                  ).astype(jnp.float32)

    for i in range(_B):
        x1 = px1[i]
        x2 = px2[i]
        y1 = py1[i]
        y2 = py2[i]
        p = conf[i]
        bsum = jnp.sum(x1) + jnp.sum(x2) + jnp.sum(y1) + jnp.sum(y2)
        skip = (bsum == 0.0) | (jnp.max(p) < 0.01)

        gts = [(gt_ref[i, g, 0], gt_ref[i, g, 1], gt_ref[i, g, 2],
                gt_ref[i, g, 3]) for g in range(_G)]

        an_g = [None] * _G
        best_c = [jnp.full((_CR, _C), -1.0, jnp.float32)] * _NC
        bestg_c = [jnp.zeros((_CR, _C), jnp.float32)] * _NC
        for g0 in range(0, _G, _GG):
            cm = [jnp.full((_CR, _C), -1.0, jnp.float32)] * _GG
            ci = [jnp.zeros((_CR, _C), jnp.float32)] * _GG
            for c in range(_NC):
                sl = pl.ds(c * _CR, _CR)
                a1 = ax1[sl]
                a2 = ax2[sl]
                b1 = ay1[sl]
                b2 = ay2[sl]
                area_a = (a2 - a1) * (b2 - b1)
                best = best_c[c]
                bestg = bestg_c[c]
                for gg in range(_GG):
                    g = g0 + gg
                    gx1, gy1, gx2, gy2 = gts[g]
                    whx = jnp.maximum(
                        jnp.minimum(a2, gx2) - jnp.maximum(a1, gx1), 0.0)
                    why = jnp.maximum(
                        jnp.minimum(b2, gy2) - jnp.maximum(b1, gy1), 0.0)
                    inter = whx * why
                    area_b = (gx2 - gx1) * (gy2 - gy1)
                    union = area_a + area_b - inter
                    iou = inter / jnp.maximum(union, 1e-9)
                    upd = iou > best
                    best = jnp.where(upd, iou, best)
                    bestg = jnp.where(upd, jnp.float32(g), bestg)
                    upd2 = iou > cm[gg]
                    cm[gg] = jnp.where(upd2, iou, cm[gg])
                    ci[gg] = jnp.where(upd2, jnp.float32(c), ci[gg])
                best_c[c] = best
                bestg_c[c] = bestg
            for gg in range(_GG):
                m = jnp.max(cm[gg])
                an_g[g0 + gg] = jnp.max(
                    jnp.where(cm[gg] == m,
                              ci[gg] * jnp.float32(-_CE) + npos_iota,
                              jnp.float32(-1e9)))

        best = jnp.concatenate(best_c, axis=0)
        bestg = jnp.concatenate(bestg_c, axis=0)

        force = jnp.zeros((_R, _C), jnp.float32)
        for g in range(_G):
            force = jnp.maximum(force,
                                jnp.where(niota == an_g[g], 1.0, 0.0))

        zero = jnp.zeros((_R, _C), jnp.float32)
        mcx = zero
        mcy = zero
        mw = zero
        mh = zero
        for g in range(_G):
            gx1, gy1, gx2, gy2 = gts[g]
            sel = bestg == jnp.float32(g)
            mcx = jnp.where(sel, (gx1 + gx2) * 0.5, mcx)
            mcy = jnp.where(sel, (gy1 + gy2) * 0.5, mcy)
            mw = jnp.where(sel, gx2 - gx1, mw)
            mh = jnp.where(sel, gy2 - gy1, mh)

        pos = (best > _IOU_TH) | (force > 0.0)
        posf = pos.astype(jnp.float32)
        np_f = jnp.sum(posf)
        np_i = np_f.astype(jnp.int32)

        ll = (_smooth_l1((x1 + x2) * 0.5 - mcx)
              + _smooth_l1((y1 + y2) * 0.5 - mcy)
              + _smooth_l1((x2 - x1) - mw)
              + _smooth_l1((y2 - y1) - mh))
        loc_i = jnp.sum(ll * posf)

        logp = jnp.maximum(jnp.log(p), -100.0)
        log1mp = jnp.maximum(jnp.log(1.0 - p), -100.0)
        pos_conf = jnp.sum(posf * (-logp))
        neg = jnp.abs(jnp.where(pos, 0.0, -log1mp))
        k = jnp.minimum(np_i * _NEG_POS_RATIO, _A - np_i)

        nbits_ref[i] = lax.bitcast_convert_type(neg, jnp.int32)
        vals = (np_f, k.astype(jnp.float32), loc_i, pos_conf,
                jnp.where(skip, jnp.float32(1.0), jnp.float32(0.0)))
        for f in range(5):
            for j in range(16):
                scal_ref[i * 5 + f, j] = vals[f]


_GD = lax.GatherDimensionNumbers(
    offset_dims=(), collapsed_slice_dims=(0,), start_index_map=(0,))


def _shuf(x, perm):
    return lax.gather(x, perm[:, None], _GD, (1,),
                      mode=lax.GatherScatterMode.PROMISE_IN_BOUNDS)


def _splat_sum(x, iota16):
    for sh in (8, 4, 2, 1):
        x = x + _shuf(x, iota16 ^ sh)
    return x


def _sc_mine_body(nbits_hbm, scal_hbm, out_hbm, row_v, scal5_v, stage_v,
                  gath_v, allscal_v, out_stage, shared):
    c = lax.axis_index("c")
    s = lax.axis_index("s")
    is_worker = (c == 0) & (s < _B)
    iota16 = lax.iota(jnp.int32, 16)
    zf = jnp.zeros((16,), jnp.float32)
    zi = jnp.zeros((16,), jnp.int32)

    @pl.when(is_worker)
    def _worker():
        pltpu.sync_copy(nbits_hbm.at[pl.ds(s * _A, _A)], row_v)
        pltpu.sync_copy(scal_hbm.at[pl.ds(s * 80, 80)], scal5_v)
        kf = scal5_v[pl.ds(16, 16)]
        ki = kf.astype(jnp.int32)

        def bit_body(j, t):
            cand = t | (zi + (jnp.int32(1) << (30 - j)))

            def cnt_body(jj, acc):
                for u in range(_UNROLL):
                    v = row_v[pl.ds(jj * (16 * _UNROLL) + u * 16, 16)]
                    acc = acc + jnp.where(v >= cand, 1, 0).astype(jnp.int32)
                return acc

            cntl = lax.fori_loop(0, _NV // _UNROLL, cnt_body, zi)
            cnt = _splat_sum(cntl, iota16)
            return jnp.where(cnt >= ki, cand, t)

        t_bits = lax.fori_loop(0, 31, bit_body, zi)
        t_val = lax.bitcast_convert_type(t_bits, jnp.float32)

        def sum_body(jj, carry):
            sg, cg = carry
            for u in range(_UNROLL):
                v = row_v[pl.ds(jj * (16 * _UNROLL) + u * 16, 16)]
                m = v > t_bits
                f = lax.bitcast_convert_type(v, jnp.float32)
                sg = sg + jnp.where(m, f, 0.0)
                cg = cg + jnp.where(m, 1.0, 0.0)
            return sg, cg

        sgl, cgl = lax.fori_loop(0, _NV // _UNROLL, sum_body, (zf, zf))
        sum_gt = _splat_sum(sgl, iota16)
        cnt_gt = _splat_sum(cgl, iota16)
        neg_conf = jnp.where(ki > 0, sum_gt + (kf - cnt_gt) * t_val, zf)
        stage_v[...] = neg_conf
        pltpu.sync_copy(stage_v, shared.at[pl.ds(s * 16, 16)])

    plsc.subcore_barrier()

    @pl.when((c == 0) & (s == 0))
    def _leader():
        pltpu.sync_copy(shared, gath_v)
        pltpu.sync_copy(scal_hbm, allscal_v)
        total_loc = zf
        total_conf = zf
        num_pos = zf
        for i in range(_B):
            np_f = allscal_v[pl.ds(i * 80, 16)]
            kf = allscal_v[pl.ds(i * 80 + 16, 16)]
            loc_i = allscal_v[pl.ds(i * 80 + 32, 16)]
            pos_conf = allscal_v[pl.ds(i * 80 + 48, 16)]
            skipf = allscal_v[pl.ds(i * 80 + 64, 16)]
            negc = gath_v[pl.ds(i * 16, 16)]
            skip = skipf > 0.5
            conf_i = (1.5 * pos_conf / jnp.maximum(np_f, 1.0)
                      + negc / jnp.maximum(kf, 1.0))
            total_conf = total_conf + jnp.where(skip, zf + 5.0, conf_i)
            total_loc = total_loc + jnp.where(skip, zf, loc_i)
            num_pos = num_pos + jnp.where(skip, zf, np_f)
        num_pos = jnp.maximum(num_pos, 1.0)
        res = (total_loc / num_pos * _LOC_W
               + total_conf * (_CONF_W / _B))
        out_stage[...] = res
        pltpu.sync_copy(out_stage, out_hbm)


@functools.cache
def _get_sc_mine():
    mesh = plsc.VectorSubcoreMesh(core_axis_name="c", subcore_axis_name="s")
    return pl.kernel(
        _sc_mine_body,
        mesh=mesh,
        compiler_params=pltpu.CompilerParams(needs_layout_passes=False),
        out_type=jax.ShapeDtypeStruct((16,), jnp.float32),
        scratch_types=[
            pltpu.VMEM((_A,), jnp.int32),
            pltpu.VMEM((80,), jnp.float32),
            pltpu.VMEM((16,), jnp.float32),
            pltpu.VMEM((_B * 16,), jnp.float32),
            pltpu.VMEM((_B * 80,), jnp.float32),
            pltpu.VMEM((16,), jnp.float32),
            pltpu.VMEM_SHARED((_B * 16,), jnp.float32),
        ],
    )


def kernel(bbox_pred, conf_pred, anchors, gt_boxes):
    px1 = bbox_pred[:, :, 0].reshape(_B, _R, _C)
    py1 = bbox_pred[:, :, 1].reshape(_B, _R, _C)
    px2 = bbox_pred[:, :, 2].reshape(_B, _R, _C)
    py2 = bbox_pred[:, :, 3].reshape(_B, _R, _C)
    conf = conf_pred.reshape(_B, _R, _C)
    ax1 = anchors[:, 0].reshape(_R, _C)
    ay1 = anchors[:, 1].reshape(_R, _C)
    ax2 = anchors[:, 2].reshape(_R, _C)
    ay2 = anchors[:, 3].reshape(_R, _C)

    nbits3, scal = pl.pallas_call(
        _dense_body,
        out_shape=(jax.ShapeDtypeStruct((_B, _R, _C), jnp.int32),
                   jax.ShapeDtypeStruct((_B * 5, 16), jnp.float32)),
        in_specs=[pl.BlockSpec(memory_space=pltpu.VMEM)] * 9
        + [pl.BlockSpec(memory_space=pltpu.SMEM)],
        out_specs=(pl.BlockSpec(memory_space=pltpu.VMEM),
                   pl.BlockSpec(memory_space=pltpu.SMEM)),
    )(px1, px2, py1, py2, conf, ax1, ax2, ay1, ay2, gt_boxes)

    nbits = nbits3.reshape(_B * _A)
    res = _get_sc_mine()(nbits, scal.reshape(_B * 5 * 16))
    return res[0]
